# baseline (speedup 1.0000x reference)
import functools

import jax
import jax.numpy as jnp
from jax import lax
from jax.experimental import pallas as pl
from jax.experimental.pallas import tpu as pltpu
from jax.experimental.pallas import tpu_sc as plsc

_DIM = 32
_SEQ = 50
_BATCH = 4096
_B = _BATCH * _SEQ
_NW = 32
_C = 128
_NB = 2

_V = 1_000_000
_CW = 512
_NCHUNK = 999936 // _CW
_TAIL0 = _NCHUNK * _CW
_NTAIL = _V - _TAIL0
_PACK = _DIM * _CW // 128

_mesh = plsc.VectorSubcoreMesh(core_axis_name="c", subcore_axis_name="s")


@functools.partial(
    pl.kernel,
    mesh=_mesh,
    out_type=(
        jax.ShapeDtypeStruct((_SEQ, _DIM // 8, _BATCH // _C, 8, _C), jnp.float32),
        jax.ShapeDtypeStruct((_V // 4, 128), jnp.float32),
    ),
    scratch_types=[
        pltpu.VMEM((_NB, _DIM, _CW), jnp.float32),
        pltpu.VMEM((_NB, _PACK, 128), jnp.float32),
        pltpu.VMEM((_NTAIL, _DIM), jnp.float32),
        pltpu.VMEM((_NTAIL * _DIM // 128, 128), jnp.float32),
        pltpu.VMEM((_SEQ * _C,), jnp.int32),
        pltpu.VMEM((_SEQ * _C,), jnp.int32),
        pltpu.VMEM((_SEQ * _C,), jnp.int32),
        pltpu.VMEM((_NB, _DIM // 8, 8, _C), jnp.float32),
        pltpu.SemaphoreType.DMA,
        pltpu.SemaphoreType.DMA,
        pltpu.SemaphoreType.DMA,
        pltpu.SemaphoreType.DMA,
        pltpu.SemaphoreType.REGULAR,
    ],
    compiler_params=pltpu.CompilerParams(
        use_tc_tiling_on_sc=True, needs_layout_passes=False
    ),
)
def _emb(
    et_hbm, etail_hbm, idx_hbm,
    out_hbm, erm_hbm,
    stage_v, pack_v, tstage_v, tpack_v, idx_v, qidx_v, jv_v, btrans_v,
    insem, outsem, gsem, wsem, xsem,
):
    core = lax.axis_index("c")
    sub = lax.axis_index("s")
    wid = sub * 2 + core
    npw = _SEQ * _C

    iota = jnp.arange(16, dtype=jnp.int32)
    iota50 = iota * _SEQ
    idiv4 = lax.shift_right_logical(iota, 2)
    colbase = lax.bitwise_and(iota, 3) * _DIM
    chi = lax.shift_right_logical(iota, 3)
    clo = lax.bitwise_and(iota, 7)

    pltpu.sync_copy(idx_hbm.at[pl.ds(pl.multiple_of(wid * npw, npw), npw)], idx_v)

    def regroup(t, carry):
        for g in range(8):
            vals = plsc.load_gather(idx_v, [iota50 + (t + 800 * g)])
            qidx_v[pl.ds(t * _C + 16 * g, 16)] = lax.shift_right_logical(vals, 2)
            jv_v[pl.ds(t * _C + 16 * g, 16)] = lax.bitwise_and(vals, 3)
        return carry

    lax.fori_loop(0, _SEQ, regroup, 0)

    base = 61 * wid + lax.min(wid, 1)
    nmine = 62 - lax.min(wid, 1)

    def fire_in(i, buf):
        col0 = pl.multiple_of(i * _CW, _CW)
        pltpu.async_copy(et_hbm.at[:, pl.ds(col0, _CW)], stage_v.at[buf], insem)

    def wait_in(buf):
        pltpu.make_async_copy(
            et_hbm.at[:, pl.ds(0, _CW)], stage_v.at[buf], insem
        ).wait()

    def wait_one_out():
        pltpu.make_async_copy(
            pack_v.at[0], erm_hbm.at[pl.ds(0, _PACK)], outsem
        ).wait()

    fire_in(base, 0)

    def chunk_step(k, carry):
        i = base + k
        buf = lax.rem(k, _NB)
        wait_in(buf)

        @pl.when(k + 1 < nmine)
        def _prefetch():
            fire_in(i + 1, lax.rem(k + 1, _NB))

        @pl.when(k >= _NB)
        def _retire():
            wait_one_out()

        st = stage_v.at[buf]
        pk = pack_v.at[buf]

        def rblock(rb, c2):
            row_vec = idiv4 + rb * 4
            for c in range(_DIM):
                g = st[c, pl.ds(rb * 16, 16)]
                plsc.store_scatter(pk, [row_vec, colbase + c], g)
            return c2

        lax.fori_loop(0, _CW // 16, rblock, 0)
        pltpu.async_copy(
            pk, erm_hbm.at[pl.ds(pl.multiple_of(i * _PACK, _PACK), _PACK)], outsem
        )
        return carry

    lax.fori_loop(0, nmine, chunk_step, 0)

    @pl.when(wid == _NW - 1)
    def _tail():
        pltpu.sync_copy(etail_hbm, tstage_v)

        def trow(r, c2):
            rowv = jnp.full((16,), lax.shift_right_logical(r, 2), jnp.int32)
            colv = lax.bitwise_and(r, 3) * _DIM + iota
            g0 = tstage_v[r, pl.ds(0, 16)]
            g1 = tstage_v[r, pl.ds(16, 16)]
            plsc.store_scatter(tpack_v, [rowv, colv], g0)
            plsc.store_scatter(tpack_v, [rowv, colv + 16], g1)
            return c2

        lax.fori_loop(0, _NTAIL, trow, 0)
        pltpu.sync_copy(
            tpack_v, erm_hbm.at[pl.ds(_TAIL0 * _DIM // 128, _NTAIL * _DIM // 128)]
        )

    def drain(k, carry):
        wait_one_out()
        return carry

    lax.fori_loop(0, lax.min(nmine, _NB), drain, 0)

    plsc.subcore_barrier()

    @pl.when(sub == 0)
    def _cross():
        pltpu.semaphore_signal(xsem, 1, core_index=1 - core)
        pl.semaphore_wait(xsem, 1)

    plsc.subcore_barrier()

    def fire_gather(t):
        pltpu.async_copy(
            erm_hbm.at[qidx_v.at[pl.ds(t * _C, _C)]],
            pack_v.at[lax.rem(t, _NB)],
            gsem,
        )

    def wait_gather(buf):
        pltpu.make_async_copy(
            erm_hbm.at[pl.ds(0, _C)], pack_v.at[buf], gsem
        ).wait()

    def wait_one_write():
        pltpu.make_async_copy(
            btrans_v.at[0], out_hbm.at[0, :, 0], wsem
        ).wait()

    fire_gather(0)

    def bstep(t, carry):
        buf = lax.rem(t, _NB)
        wait_gather(buf)

        @pl.when(t + 1 < _SEQ)
        def _fire_next():
            fire_gather(t + 1)

        @pl.when(t >= _NB)
        def _retire():
            wait_one_write()

        rows = pack_v.at[buf]
        tr = btrans_v.at[buf]

        def extract(g, c2):
            rowv = iota + 16 * g
            jm = jv_v[pl.ds(t * _C + 16 * g, 16)] * _DIM
            for c in range(_DIM):
                vals = plsc.load_gather(rows, [rowv, jm + c])
                plsc.store_scatter(
                    tr,
                    [
                        jnp.full((16,), c // 8, jnp.int32),
                        jnp.full((16,), c % 8, jnp.int32),
                        rowv,
                    ],
                    vals,
                )
            return c2

        lax.fori_loop(0, 8, extract, 0)
        pltpu.async_copy(btrans_v.at[buf], out_hbm.at[t, :, wid], wsem)
        return carry

    lax.fori_loop(0, _SEQ, bstep, 0)

    for _ in range(_NB):
        wait_one_write()


@jax.jit
def kernel(inputs, e):
    idx = inputs.reshape(_B).astype(jnp.int32)
    et = e.T
    etail = e[_TAIL0:, :]
    out5, _ = _emb(et, etail, idx)
    return out5.transpose(2, 4, 0, 1, 3).reshape(_BATCH, _SEQ, _DIM)


# speedup vs baseline: 1.2075x; 1.2075x over previous
import functools

import jax
import jax.numpy as jnp
from jax import lax
from jax.experimental import pallas as pl
from jax.experimental.pallas import tpu as pltpu
from jax.experimental.pallas import tpu_sc as plsc

_DIM = 32
_SEQ = 50
_BATCH = 4096
_B = _BATCH * _SEQ
_NW = 32
_C = 128
_NB = 2

_mesh = plsc.VectorSubcoreMesh(core_axis_name="c", subcore_axis_name="s")


@functools.partial(
    pl.kernel,
    mesh=_mesh,
    out_type=jax.ShapeDtypeStruct((_SEQ, _DIM // 8, _BATCH // _C, 8, _C), jnp.float32),
    scratch_types=[
        pltpu.VMEM((_SEQ * _C,), jnp.int32),
        pltpu.VMEM((_SEQ * _C,), jnp.int32),
        pltpu.VMEM((3, _C, _DIM), jnp.float32),
        pltpu.VMEM((_NB, _DIM // 8, 8, _C), jnp.float32),
        pltpu.SemaphoreType.DMA,
        pltpu.SemaphoreType.DMA,
    ],
    compiler_params=pltpu.CompilerParams(use_tc_tiling_on_sc=False, needs_layout_passes=False),
)
def _emb(table_hbm, idx_hbm, out_hbm, idx_v, cidx_v, rows_v, trans_v, gsem, wsem):
    wid = lax.axis_index("s") * 2 + lax.axis_index("c")
    npw = _SEQ * _C
    pltpu.sync_copy(idx_hbm.at[pl.ds(pl.multiple_of(wid * npw, npw), npw)], idx_v)

    iota = jnp.arange(16, dtype=jnp.int32)
    iota50 = iota * _SEQ
    iota128 = iota * _C
    chi = lax.shift_right_logical(iota, 3)
    clo = lax.bitwise_and(iota, 7)

    def regroup(t, carry):
        for g in range(8):
            vals = plsc.load_gather(idx_v, [iota50 + (t + 800 * g)])
            cidx_v[pl.ds(t * _C + 16 * g, 16)] = vals
        return carry

    lax.fori_loop(0, _SEQ, regroup, 0)

    def fire_gather(t):
        pltpu.async_copy(
            table_hbm.at[cidx_v.at[pl.ds(t * _C, _C)]],
            rows_v.at[lax.rem(t, 3)],
            gsem,
        )

    def wait_gather(b):
        pltpu.make_async_copy(
            table_hbm.at[pl.ds(0, _C)], rows_v.at[b], gsem
        ).wait()

    def wait_one_write():
        pltpu.make_async_copy(
            trans_v.at[0], out_hbm.at[0, :, 0], wsem
        ).wait()

    fire_gather(0)
    fire_gather(1)

    def step(t, carry):
        b = lax.rem(t, 3)
        wait_gather(b)

        @pl.when(t + 2 < _SEQ)
        def _fire_next():
            fire_gather(t + 2)

        @pl.when(t >= _NB)
        def _retire():
            wait_one_write()

        rows = rows_v.at[b]
        tr = trans_v.at[lax.rem(t, _NB)]

        def tpose(g, c2):
            rowv = iota + 16 * g
            for c in range(_DIM):
                vals = plsc.load_gather(rows, [rowv, jnp.full((16,), c, jnp.int32)])
                plsc.store_scatter(
                    tr,
                    [
                        jnp.full((16,), c // 8, jnp.int32),
                        jnp.full((16,), c % 8, jnp.int32),
                        rowv,
                    ],
                    vals,
                )
            return c2

        lax.fori_loop(0, 8, tpose, 0)
        pltpu.async_copy(trans_v.at[lax.rem(t, _NB)], out_hbm.at[t, :, wid], wsem)
        return carry

    lax.fori_loop(0, _SEQ, step, 0)

    for _ in range(_NB):
        wait_one_write()


@jax.jit
def kernel(inputs, e):
    idx = inputs.reshape(_B).astype(jnp.int32)
    out5 = _emb(e, idx)
    return out5.transpose(2, 4, 0, 1, 3).reshape(_BATCH, _SEQ, _DIM)


# speedup vs baseline: 1.2724x; 1.0537x over previous
import functools

import jax
import jax.numpy as jnp
from jax import lax
from jax.experimental import pallas as pl
from jax.experimental.pallas import tpu as pltpu
from jax.experimental.pallas import tpu_sc as plsc

_DIM = 32
_SEQ = 50
_BATCH = 4096
_B = _BATCH * _SEQ
_NW = 32
_C = 128
_NB = 2

_mesh = plsc.VectorSubcoreMesh(core_axis_name="c", subcore_axis_name="s")


@functools.partial(
    pl.kernel,
    mesh=_mesh,
    out_type=jax.ShapeDtypeStruct((_SEQ, _DIM // 8, _BATCH // _C, 8, _C), jnp.float32),
    scratch_types=[
        pltpu.VMEM((_SEQ * _C,), jnp.int32),
        pltpu.VMEM((_SEQ * _C,), jnp.int32),
        pltpu.VMEM((3, _C, _DIM), jnp.float32),
        pltpu.VMEM((_NB, _DIM // 8, 8, _C), jnp.float32),
        pltpu.SemaphoreType.DMA,
        pltpu.SemaphoreType.DMA,
    ],
    compiler_params=pltpu.CompilerParams(use_tc_tiling_on_sc=False, needs_layout_passes=False),
)
def _emb(table_hbm, idx_hbm, out_hbm, idx_v, cidx_v, rows_v, trans_v, gsem, wsem):
    wid = lax.axis_index("s") * 2 + lax.axis_index("c")
    npw = _SEQ * _C
    pltpu.sync_copy(idx_hbm.at[pl.ds(pl.multiple_of(wid * npw, npw), npw)], idx_v)

    iota = jnp.arange(16, dtype=jnp.int32)
    iota50 = iota * _SEQ
    iota128 = iota * _C
    chi = lax.shift_right_logical(iota, 3)
    clo = lax.bitwise_and(iota, 7)

    def regroup(t, carry):
        for g in range(8):
            vals = plsc.load_gather(idx_v, [iota50 + (t + 800 * g)])
            cidx_v[pl.ds(t * _C + 16 * g, 16)] = vals
        return carry

    lax.fori_loop(0, _SEQ, regroup, 0)

    def fire_gather(t):
        pltpu.async_copy(
            table_hbm.at[cidx_v.at[pl.ds(t * _C, _C)]],
            rows_v.at[lax.rem(t, 3)],
            gsem,
        )

    def wait_gather(b):
        pltpu.make_async_copy(
            table_hbm.at[pl.ds(0, _C)], rows_v.at[b], gsem
        ).wait()

    def wait_one_write():
        pltpu.make_async_copy(
            trans_v.at[0], out_hbm.at[0, :, 0], wsem
        ).wait()

    fire_gather(0)
    fire_gather(1)

    def step(t, carry):
        b = lax.rem(t, 3)
        wait_gather(b)

        @pl.when(t + 2 < _SEQ)
        def _fire_next():
            fire_gather(t + 2)

        @pl.when(t >= _NB)
        def _retire():
            wait_one_write()

        rows = rows_v.at[b]
        tr = trans_v.at[lax.rem(t, _NB)]

        def tpose(q, c2):
            for u in range(4):
                bl = q * 4 + u
                blv = jnp.full((16,), bl, jnp.int32)
                g0 = rows[bl, pl.ds(0, 16)]
                g1 = rows[bl, pl.ds(16, 16)]
                plsc.store_scatter(tr, [chi, clo, blv], g0)
                plsc.store_scatter(tr, [chi + 2, clo, blv], g1)
            return c2

        lax.fori_loop(0, _C // 4, tpose, 0)
        pltpu.async_copy(trans_v.at[lax.rem(t, _NB)], out_hbm.at[t, :, wid], wsem)
        return carry

    lax.fori_loop(0, _SEQ, step, 0)

    for _ in range(_NB):
        wait_one_write()


@jax.jit
def kernel(inputs, e):
    idx = inputs.reshape(_B).astype(jnp.int32)
    out5 = _emb(e, idx)
    return out5.transpose(2, 4, 0, 1, 3).reshape(_BATCH, _SEQ, _DIM)
